# baseline (speedup 1.0000x reference)
import functools

import jax
import jax.numpy as jnp
from jax import lax
from jax.experimental import pallas as pl
from jax.experimental.pallas import tpu as pltpu
from jax.experimental.pallas import tpu_sc as plsc

N = 16384
P = 20
D = 32
R = N * P
NC, NS = 2, 16
NW = NC * NS
ROWS_W = R // NW
G = 128
GW = ROWS_W // G
CHUNK = 512
NG = CHUNK // G
NCHUNK = ROWS_W // CHUNK
NT = NCHUNK // 2


def _sc_embed(road_table, datetime_table, idx_road, idx_dt):
  mesh = plsc.VectorSubcoreMesh(core_axis_name="c", subcore_axis_name="s")

  @functools.partial(
      pl.kernel,
      mesh=mesh,
      compiler_params=pltpu.CompilerParams(use_tc_tiling_on_sc=False),
      out_type=jax.ShapeDtypeStruct((R, 2, D), jnp.float32),
      scratch_types=[
          pltpu.VMEM((GW, G), jnp.int32),
          pltpu.VMEM((GW, G), jnp.int32),
          pltpu.VMEM((2, CHUNK, D), jnp.float32),
          pltpu.VMEM((2, CHUNK, D), jnp.float32),
          pltpu.SemaphoreType.DMA,
          pltpu.SemaphoreType.DMA,
          pltpu.SemaphoreType.DMA,
          pltpu.SemaphoreType.DMA,
      ],
  )
  def k(road_hbm, dt_hbm, idxr_hbm, idxd_hbm, out_hbm,
        idxr_v, idxd_v, rbuf, dbuf, gsem0, gsem1, wsem0, wsem1):
    wid = lax.axis_index("s") * NC + lax.axis_index("c")
    w0 = pl.multiple_of(wid * GW, GW)
    pltpu.sync_copy(idxr_hbm.at[pl.ds(w0, GW)], idxr_v)
    pltpu.sync_copy(idxd_hbm.at[pl.ds(w0, GW)], idxd_v)
    gsem = (gsem0, gsem1)
    wsem = (wsem0, wsem1)

    def issue_gathers(ci, p):
      for g in range(NG):
        gi = ci * NG + g
        pltpu.async_copy(road_hbm.at[idxr_v.at[gi]],
                         rbuf.at[p, pl.ds(g * G, G)], gsem[p])
        pltpu.async_copy(dt_hbm.at[idxd_v.at[gi]],
                         dbuf.at[p, pl.ds(g * G, G)], gsem[p])

    def wait_gathers(p):
      for g in range(NG):
        pltpu.make_async_copy(road_hbm.at[idxr_v.at[g]],
                              rbuf.at[p, pl.ds(g * G, G)], gsem[p]).wait()
        pltpu.make_async_copy(dt_hbm.at[idxd_v.at[g]],
                              dbuf.at[p, pl.ds(g * G, G)], gsem[p]).wait()

    def issue_writes(ci, p):
      row0 = pl.multiple_of(wid * ROWS_W + ci * CHUNK, CHUNK)
      pltpu.async_copy(rbuf.at[p], out_hbm.at[pl.ds(row0, CHUNK), 0], wsem[p])
      pltpu.async_copy(dbuf.at[p], out_hbm.at[pl.ds(row0, CHUNK), 1], wsem[p])

    def wait_writes(p):
      row0 = pl.multiple_of(wid * ROWS_W, CHUNK)
      pltpu.make_async_copy(rbuf.at[p], out_hbm.at[pl.ds(row0, CHUNK), 0],
                            wsem[p]).wait()
      pltpu.make_async_copy(dbuf.at[p], out_hbm.at[pl.ds(row0, CHUNK), 1],
                            wsem[p]).wait()

    issue_gathers(0, 0)

    def body(t, carry):
      c0 = 2 * t
      pl.when(t > 0)(lambda: wait_writes(1))
      issue_gathers(c0 + 1, 1)
      wait_gathers(0)
      issue_writes(c0, 0)

      def refill0():
        wait_writes(0)
        issue_gathers(c0 + 2, 0)
      pl.when(t + 1 < NT)(refill0)
      wait_gathers(1)
      issue_writes(c0 + 1, 1)
      return carry

    lax.fori_loop(0, NT, body, 0)
    wait_writes(0)
    wait_writes(1)

  return k(road_table, datetime_table, idx_road, idx_dt)


def kernel(x_road, x_datetime, road_table, datetime_table):
  xr = x_road.reshape(N).astype(jnp.int32)
  idx_road = jnp.broadcast_to(xr[:, None], (N, P)).reshape(R // G, G)
  idx_dt = x_datetime.reshape(R // G, G).astype(jnp.int32)
  out = _sc_embed(road_table, datetime_table, idx_road, idx_dt)
  return out.reshape(N, P, 2 * D)
